# Initial kernel scaffold; baseline (speedup 1.0000x reference)
import jax
import jax.numpy as jnp
from jax.experimental import pallas as pl


def kernel(x, forward_level, gate, rc_pair_index, params):
    raise NotImplementedError("write your pallas kernel here")


# speedup vs baseline: 14.9708x; 14.9708x over previous
import functools

import jax
import jax.numpy as jnp
import numpy as np
from jax import lax
from jax.experimental import pallas as pl
from jax.experimental.pallas import tpu as pltpu
from jax.experimental.pallas import tpu_sc as plsc

_H = 128
_NUM_LEVELS = 8
_F32 = jnp.float32


def _dot(a, b):
    return jnp.dot(a, b, preferred_element_type=_F32)


def _fold_mlp(p, bn=False, func=False):
    W1, b1 = p['W1'], p['b1']
    if func:
        b1 = b1 + W1[:, _H:].sum(axis=1)
        W1 = W1[:, :_H]
    W2, b2, W3, b3 = p['W2'], p['b2'], p['W3'], p['b3']
    if bn:
        inv = np.float32(1.0 / np.sqrt(1.0 + 1e-5))
        s1 = p['g1'] * inv
        b1 = s1 * b1 + p['be1']
        W1 = W1 * s1[:, None]
        s2 = p['g2'] * inv
        b2 = s2 * b2 + p['be2']
        W2 = W2 * s2[:, None]
    return W1.T, b1, W2.T, b2, W3.T, b3


def _fold_pipe(p_mlp, p_gru, func=False):
    W1, b1, W2, b2, W3, b3 = _fold_mlp(p_mlp, func=func)
    Wih, bih = p_gru['W_ih'], p_gru['b_ih']
    ghc = p_gru['W_hh'].sum(axis=1) + p_gru['b_hh']
    beta = bih + Wih @ b3
    beta = beta.at[:2 * _H].add(ghc[:2 * _H])
    cn = ghc[2 * _H:]
    return W1, b1, W2, b2, W3, Wih.T, beta, cn


def _fold_all(params):
    pipes = [
        _fold_pipe(params['aggr_and_strc'], params['update_and_strc']),
        _fold_pipe(params['aggr_not_strc'], params['update_not_strc']),
        _fold_pipe(params['aggr_and_func'], params['update_and_func'], func=True),
        _fold_pipe(params['aggr_not_func'], params['update_not_func'], func=True),
    ]
    W1 = jnp.stack([p[0] for p in pipes])
    b1 = jnp.stack([p[1] for p in pipes])[:, None, :]
    W2 = jnp.stack([p[2] for p in pipes])
    b2 = jnp.stack([p[3] for p in pipes])[:, None, :]
    W3 = jnp.stack([p[4] for p in pipes])
    Wih = jnp.stack([p[5] for p in pipes])
    beta = jnp.stack([p[6] for p in pipes])[:, None, :]
    cn = jnp.stack([p[7] for p in pipes])[:, None, :]

    Wp1, bp1, Wp2, bp2, Wp3, bp3 = _fold_mlp(params['readout_prob'], bn=True)
    prob_w = (Wp1, bp1[None, :], Wp2, bp2[None, :], Wp3, bp3[None, :])

    Wr1, br1, Wr2, br2, Wr3, br3 = _fold_mlp(params['readout_rc'], bn=True)
    rc_w = (Wr1[:_H], Wr1[_H:], br1[None, :], Wr2, br2[None, :], Wr3, br3[None, :])
    return (W1, b1, W2, b2, W3, Wih, beta, cn), prob_w, rc_w


def _main_body(x_ref, fl_ref, g_ref,
               W1_ref, b1_ref, W2_ref, b2_ref, W3_ref, Wih_ref, beta_ref, cn_ref,
               Wp1_ref, bp1_ref, Wp2_ref, bp2_ref, Wp3_ref, bp3_ref,
               hs_ref, hf_ref, prob_ref):
    xb = x_ref[...]
    fl = fl_ref[...]
    g = g_ref[...]
    act = (fl >= 1) & (fl <= _NUM_LEVELS - 1)
    m_and = act & (g == 1)
    m_not = act & (g == 2)

    outs = []
    for t in range(4):
        h = jnp.maximum(_dot(xb, W1_ref[t]) + b1_ref[t], 0.0)
        h = jnp.maximum(_dot(h, W2_ref[t]) + b2_ref[t], 0.0)
        msg = _dot(h, W3_ref[t])
        gi = _dot(msg, Wih_ref[t]) + beta_ref[t]
        r = jax.nn.sigmoid(gi[:, :_H])
        z = jax.nn.sigmoid(gi[:, _H:2 * _H])
        n = jnp.tanh(gi[:, 2 * _H:] + r * cn_ref[t])
        outs.append((1.0 - z) * n + z)

    hs = jnp.where(m_and, outs[0], jnp.where(m_not, outs[1], 1.0))
    hf = jnp.where(m_and, outs[2], jnp.where(m_not, outs[3], 1.0))
    hs_ref[...] = hs
    hf_ref[...] = hf

    ph = jnp.maximum(_dot(hf, Wp1_ref[...]) + bp1_ref[...], 0.0)
    ph = jnp.maximum(_dot(ph, Wp2_ref[...]) + bp2_ref[...], 0.0)
    prob_ref[...] = _dot(ph, Wp3_ref[...]) + bp3_ref[...]


def _full_spec(shape):
    nd = len(shape)
    return pl.BlockSpec(shape, lambda i, _nd=nd: (0,) * _nd)


def _main_call(x, fl2, g2, pipe_w, prob_w, block_n):
    n = x.shape[0]
    grid = (n // block_n,)
    W1, b1, W2, b2, W3, Wih, beta, cn = pipe_w
    Wp1, bp1, Wp2, bp2, Wp3, bp3 = prob_w
    in_specs = [
        pl.BlockSpec((block_n, _H), lambda i: (i, 0)),
        pl.BlockSpec((block_n, 1), lambda i: (i, 0)),
        pl.BlockSpec((block_n, 1), lambda i: (i, 0)),
    ] + [_full_spec(w.shape) for w in
         (W1, b1, W2, b2, W3, Wih, beta, cn, Wp1, bp1, Wp2, bp2, Wp3, bp3)]
    out_specs = [
        pl.BlockSpec((block_n, _H), lambda i: (i, 0)),
        pl.BlockSpec((block_n, _H), lambda i: (i, 0)),
        pl.BlockSpec((block_n, 1), lambda i: (i, 0)),
    ]
    out_shape = [
        jax.ShapeDtypeStruct((n, _H), _F32),
        jax.ShapeDtypeStruct((n, _H), _F32),
        jax.ShapeDtypeStruct((n, 1), _F32),
    ]
    return pl.pallas_call(
        _main_body,
        grid=grid,
        in_specs=in_specs,
        out_specs=out_specs,
        out_shape=out_shape,
        compiler_params=pltpu.CompilerParams(
            dimension_semantics=("arbitrary",)),
    )(x, fl2, g2, W1, b1, W2, b2, W3, Wih, beta, cn,
      Wp1, bp1, Wp2, bp2, Wp3, bp3)


@functools.cache
def _make_sc_gather(num_rows, d):
    info = plsc.get_sparse_core_info()
    nw = info.num_cores * info.num_subcores
    b_per_w = num_rows // nw
    mesh = plsc.VectorSubcoreMesh(core_axis_name="c", subcore_axis_name="s")

    @functools.partial(
        pl.kernel,
        out_type=jax.ShapeDtypeStruct((num_rows, d), _F32),
        mesh=mesh,
        scratch_types=[
            pltpu.VMEM((b_per_w,), jnp.int32),
            pltpu.VMEM((b_per_w, d), _F32),
            pltpu.SemaphoreType.DMA,
        ],
    )
    def gather(table_hbm, idx_hbm, out_hbm, idx_v, rows_v, sem):
        wid = lax.axis_index("s") * info.num_cores + lax.axis_index("c")
        base = wid * b_per_w
        pltpu.sync_copy(idx_hbm.at[pl.ds(base, b_per_w)], idx_v)
        pltpu.async_copy(table_hbm.at[idx_v], rows_v, sem).wait()
        pltpu.sync_copy(rows_v, out_hbm.at[pl.ds(base, b_per_w)])

    return gather


def _rc_body(u_ref, v_ref, A1_ref, B1_ref, b1_ref, W2_ref, b2_ref,
             W3_ref, b3_ref, out_ref):
    h = _dot(u_ref[...], A1_ref[...]) + _dot(v_ref[...], B1_ref[...]) + b1_ref[...]
    h = jnp.maximum(h, 0.0)
    h = jnp.maximum(_dot(h, W2_ref[...]) + b2_ref[...], 0.0)
    out_ref[...] = jax.nn.sigmoid(_dot(h, W3_ref[...]) + b3_ref[...])


def _rc_call(u, v, rc_w, block_p):
    p = u.shape[0]
    A1, B1, b1, W2, b2, W3, b3 = rc_w
    grid = (p // block_p,)
    in_specs = [
        pl.BlockSpec((block_p, _H), lambda i: (i, 0)),
        pl.BlockSpec((block_p, _H), lambda i: (i, 0)),
    ] + [_full_spec(w.shape) for w in (A1, B1, b1, W2, b2, W3, b3)]
    return pl.pallas_call(
        _rc_body,
        grid=grid,
        in_specs=in_specs,
        out_specs=pl.BlockSpec((block_p, 1), lambda i: (i, 0)),
        out_shape=jax.ShapeDtypeStruct((p, 1), _F32),
        compiler_params=pltpu.CompilerParams(
            dimension_semantics=("arbitrary",)),
    )(u, v, A1, B1, b1, W2, b2, W3, b3)


def _pick_block(n, target):
    b = min(target, n)
    while n % b or b % 8:
        b -= 8 if b % 8 == 0 else b % 8
        if b <= 8:
            return 8
    return b


def kernel(x, forward_level, gate, rc_pair_index, params):
    n = x.shape[0]
    p = rc_pair_index.shape[1]
    pipe_w, prob_w, rc_w = _fold_all(params)
    fl2 = forward_level.astype(jnp.int32).reshape(n, 1)
    g2 = gate.astype(jnp.int32).reshape(n, 1)

    block_n = _pick_block(n, 2000)
    hs, hf, prob = _main_call(x, fl2, g2, pipe_w, prob_w, block_n)

    info = plsc.get_sparse_core_info()
    align = 8 * info.num_cores * info.num_subcores
    idx = rc_pair_index.astype(jnp.int32).reshape(-1)
    pad = (-idx.shape[0]) % align
    if pad:
        idx = jnp.pad(idx, (0, pad))
    rows = _make_sc_gather(idx.shape[0], _H)(hs, idx)
    u = rows[:p]
    v = rows[p:2 * p]

    block_p = _pick_block(p, 2000)
    is_rc = _rc_call(u, v, rc_w, block_p)
    return (hs, hf, prob, is_rc)
